# baseline (speedup 1.0000x reference)
import functools

import jax
import jax.numpy as jnp
from jax import lax
from jax.experimental import pallas as pl
from jax.experimental.pallas import tpu as pltpu
from jax.experimental.pallas import tpu_sc as plsc

_NUM_EMBEDDINGS = 1000000
_MIN_SCALE = 2.5
_MAX_SCALE = 3.5

_ROWS = 16384
_COLS = 200
_TOTAL = _ROWS * _COLS
_W = 128
_H = _TOTAL // _W

_info = plsc.get_sparse_core_info()
_NC = _info.num_cores
_NS = _info.num_subcores
_NW = _NC * _NS
_PER_W = _H // _NW

_R = 16
_STEPS = _PER_W // _R
_OUTER = _STEPS // 2

_mesh = plsc.VectorSubcoreMesh(core_axis_name="c", subcore_axis_name="s")


@functools.partial(
    pl.kernel,
    mesh=_mesh,
    out_type=jax.ShapeDtypeStruct((_H, _W), jnp.float32),
    scratch_types=(
        [pltpu.VMEM((_R, _W), jnp.float32) for _ in range(2)]
        + [pltpu.VMEM((_R, _W), jnp.int32) for _ in range(2)]
        + [pltpu.VMEM((_R, _W), jnp.int32) for _ in range(2)]
        + [pltpu.VMEM((_R, _W), jnp.float32) for _ in range(2)]
        + [pltpu.VMEM((_R, _W), jnp.float32) for _ in range(2)]
        + [pltpu.SemaphoreType.DMA, pltpu.SemaphoreType.DMA,
           pltpu.SemaphoreType.DMA, pltpu.SemaphoreType.DMA]
    ),
)
def _sc_remap(o3_hbm, table_hbm, pin2d_hbm, out_hbm,
              o3_0, o3_1, ilo_0, ilo_1, ihi_0, ihi_1,
              lo_0, lo_1, hi_0, hi_1, sem_g0, sem_g1, sem_o0, sem_o1):
    del pin2d_hbm
    o3_v = (o3_0, o3_1)
    ilo_v = (ilo_0, ilo_1)
    ihi_v = (ihi_0, ihi_1)
    lo_v = (lo_0, lo_1)
    hi_v = (hi_0, hi_1)
    sem_g = (sem_g0, sem_g1)
    sem_o = (sem_o0, sem_o1)

    wid = lax.axis_index("s") * _NC + lax.axis_index("c")
    base = wid * _PER_W

    def rows(i):
        return pl.ds(base + i * _R, _R)

    def derive(b):
        def body(j, c):
            r = j // 8
            sl = pl.ds((j % 8) * 16, 16)
            o3 = o3_v[b][r, sl]
            li = o3.astype(jnp.int32)
            lf = li.astype(jnp.float32)
            fr = o3 - lf
            ilo_v[b][r, sl] = li
            ihi_v[b][r, sl] = li + jnp.where(fr > 0.0, 1, 0)
            o3_v[b][r, sl] = fr
            return c
        lax.fori_loop(0, _R * 8, body, 0, unroll=True)

    def fire(b):
        for r in range(_R):
            pltpu.async_copy(table_hbm.at[ilo_v[b].at[r]], lo_v[b].at[r],
                             sem_g[b])
            pltpu.async_copy(table_hbm.at[ihi_v[b].at[r]], hi_v[b].at[r],
                             sem_g[b])

    def drain_gathers(i, b):
        pltpu.make_async_copy(o3_hbm.at[rows(i), :], lo_v[b], sem_g[b]).wait()
        pltpu.make_async_copy(o3_hbm.at[rows(i), :], hi_v[b], sem_g[b]).wait()

    def interp(b):
        def body(j, c):
            r = j // 8
            sl = pl.ds((j % 8) * 16, 16)
            fr = o3_v[b][r, sl]
            o3_v[b][r, sl] = fr * lo_v[b][r, sl] + (1.0 - fr) * hi_v[b][r, sl]
            return c
        lax.fori_loop(0, _R * 8, body, 0, unroll=True)

    def outer(g, carry):
        for b in range(2):
            i = 2 * g + b

            @pl.when(i >= 2)
            def _():
                pltpu.make_async_copy(
                    o3_hbm.at[rows(i - 2), :], o3_v[b], sem_o[b]).wait()

            pltpu.sync_copy(o3_hbm.at[rows(i), :], o3_v[b])
            derive(b)
            fire(b)

            @pl.when(i >= 1)
            def _():
                pb = 1 - b
                drain_gathers(i - 1, pb)
                interp(pb)
                pltpu.async_copy(o3_v[pb], out_hbm.at[rows(i - 1), :],
                                 sem_o[pb])

        return carry

    lax.fori_loop(0, _OUTER, outer, 0)

    last = _STEPS - 1
    lb = last % 2
    drain_gathers(last, lb)
    interp(lb)
    pltpu.sync_copy(o3_v[lb], out_hbm.at[rows(last), :])
    pltpu.make_async_copy(o3_hbm.at[rows(last - 1), :], o3_v[1 - lb],
                          sem_o[1 - lb]).wait()


def kernel(x, table, scale):
    s = jnp.clip(scale, _MIN_SCALE, _MAX_SCALE)
    mean = jnp.mean(x)
    std = jnp.std(x, ddof=1)
    out = (x - mean) / std
    out_01 = (jnp.clip(out, -s, s) / s + 1.0) / 2.0
    out3 = out_01 * (_NUM_EMBEDDINGS - 1)
    res = _sc_remap(out3.reshape(_H, _W), table.reshape(-1), out3)
    return res.reshape(_ROWS, _COLS)
